# Initial kernel scaffold; baseline (speedup 1.0000x reference)
import jax
import jax.numpy as jnp
from jax.experimental import pallas as pl


def kernel(x, edge_index, Wl, bl, Wr, br, att, gat_bias, W_nmu, b_nmu, W_nlv, b_nlv, ln1_g, ln1_b, W1, b1, ln2_g, ln2_b, W2, b2, ln3_g, ln3_b, W3, b3, W_amu, b_amu, W_alv, b_alv):
    raise NotImplementedError("write your pallas kernel here")


# speedup vs baseline: 1.5780x; 1.5780x over previous
import functools

import jax
import jax.numpy as jnp
import numpy as np
from jax.experimental import pallas as pl
from jax.experimental.pallas import tpu as pltpu

N = 10000
E = 320000
D = 128
OUT = 128
MLP_IN = OUT * 8
H1 = OUT * 2
N_PAD = 10240


def _dense_pre_body(x_ref, wl_ref, bl_ref, wr_ref, br_ref, xl_ref, xr_ref):
    x = x_ref[...]
    xl_ref[...] = jnp.dot(x, wl_ref[...], preferred_element_type=jnp.float32) + bl_ref[...]
    xr_ref[...] = jnp.dot(x, wr_ref[...], preferred_element_type=jnp.float32) + br_ref[...]


def _dense_pre(x_eps, Wl, bl, Wr, br):
    return pl.pallas_call(
        _dense_pre_body,
        out_shape=(
            jax.ShapeDtypeStruct((N, OUT), jnp.float32),
            jax.ShapeDtypeStruct((N, OUT), jnp.float32),
        ),
    )(x_eps, Wl, bl.reshape(1, OUT), Wr, br.reshape(1, OUT))


def _dense_post_body(h_ref, gb_ref, wmu_ref, bmu_ref, wlv_ref, blv_ref,
                     mu_ref, lv_ref):
    h = jnp.maximum(h_ref[...] + gb_ref[...], 0.0)
    mu_ref[...] = jnp.dot(h, wmu_ref[...], preferred_element_type=jnp.float32) + bmu_ref[...]
    lv_ref[...] = jnp.dot(h, wlv_ref[...], preferred_element_type=jnp.float32) + blv_ref[...]


def _dense_post(h_raw, gat_bias, W_nmu, b_nmu, W_nlv, b_nlv):
    return pl.pallas_call(
        _dense_post_body,
        out_shape=(
            jax.ShapeDtypeStruct((N, OUT), jnp.float32),
            jax.ShapeDtypeStruct((N, OUT), jnp.float32),
        ),
    )(h_raw, gat_bias.reshape(1, OUT), W_nmu, b_nmu.reshape(1, OUT),
      W_nlv, b_nlv.reshape(1, OUT))


@functools.lru_cache(maxsize=1)
def _pool_matrix():
    i = np.arange(MLP_IN)
    starts = (i * N) // MLP_IN
    ends = ((i + 1) * N + MLP_IN - 1) // MLP_IN
    P = np.zeros((N_PAD, MLP_IN), np.float32)
    for j in range(MLP_IN):
        P[starts[j]:ends[j], j] = 1.0 / (ends[j] - starts[j])
    return jnp.asarray(P)


def _pool_body(xT_ref, p_ref, t_ref):
    @pl.when(pl.program_id(0) == 0)
    def _init():
        t_ref[...] = jnp.zeros_like(t_ref)
    t_ref[...] += jnp.dot(xT_ref[...], p_ref[...], preferred_element_type=jnp.float32)


def _pool(xT):
    KC = 8
    KB = N_PAD // KC
    return pl.pallas_call(
        _pool_body,
        grid=(KC,),
        in_specs=[
            pl.BlockSpec((D, KB), lambda k: (0, k)),
            pl.BlockSpec((KB, MLP_IN), lambda k: (k, 0)),
        ],
        out_specs=pl.BlockSpec((D, MLP_IN), lambda k: (0, 0)),
        out_shape=jax.ShapeDtypeStruct((D, MLP_IN), jnp.float32),
    )(xT, _pool_matrix())


def _ln(x, g, b):
    m = jnp.mean(x, axis=-1, keepdims=True)
    v = jnp.mean((x - m) ** 2, axis=-1, keepdims=True)
    return (x - m) * jax.lax.rsqrt(v + 1e-5) * g + b


def _attr_mlp_body(t_ref, ln1g, ln1b, w1, b1, ln2g, ln2b, w2, b2,
                   ln3g, ln3b, w3, b3, wamu, bamu, walv, balv,
                   mu_ref, lv_ref):
    t = t_ref[...]
    t = jnp.tanh(jnp.dot(_ln(t, ln1g[...], ln1b[...]), w1[...],
                         preferred_element_type=jnp.float32) + b1[...])
    t = jnp.tanh(jnp.dot(_ln(t, ln2g[...], ln2b[...]), w2[...],
                         preferred_element_type=jnp.float32) + b2[...])
    h_A = jnp.tanh(jnp.dot(_ln(t, ln3g[...], ln3b[...]), w3[...],
                           preferred_element_type=jnp.float32) + b3[...])
    mu_ref[...] = jnp.dot(h_A, wamu[...], preferred_element_type=jnp.float32) + bamu[...]
    lv_ref[...] = jnp.dot(h_A, walv[...], preferred_element_type=jnp.float32) + balv[...]


def _attr_mlp(t, ln1_g, ln1_b, W1, b1, ln2_g, ln2_b, W2, b2,
              ln3_g, ln3_b, W3, b3, W_amu, b_amu, W_alv, b_alv):
    r = lambda a: a.reshape(1, -1)
    return pl.pallas_call(
        _attr_mlp_body,
        out_shape=(
            jax.ShapeDtypeStruct((D, OUT), jnp.float32),
            jax.ShapeDtypeStruct((D, OUT), jnp.float32),
        ),
    )(t, r(ln1_g), r(ln1_b), W1, r(b1), r(ln2_g), r(ln2_b), W2, r(b2),
      r(ln3_g), r(ln3_b), W3, r(b3), W_amu, r(b_amu), W_alv, r(b_alv))


def kernel(x, edge_index, Wl, bl, Wr, br, att, gat_bias, W_nmu, b_nmu,
           W_nlv, b_nlv, ln1_g, ln1_b, W1, b1, ln2_g, ln2_b, W2, b2,
           ln3_g, ln3_b, W3, b3, W_amu, b_amu, W_alv, b_alv):
    nkey = jax.random.key(42)
    noise1 = jax.random.normal(jax.random.fold_in(nkey, 0), x.shape, x.dtype)
    noise2 = jax.random.normal(jax.random.fold_in(nkey, 1), x.shape, x.dtype)

    x_eps = x + noise1
    xl, xr = _dense_pre(x_eps, Wl, bl, Wr, br)

    src = edge_index[0]
    dst = edge_index[1]

    e = xl[src] + xr[dst]
    e = jnp.where(e >= 0, e, 0.2 * e)
    logits = e @ att
    a = jnp.exp(logits)
    den = jax.ops.segment_sum(a, dst, num_segments=N)
    alpha = a / (den[dst] + 1e-16)
    h_raw = jax.ops.segment_sum(xl[src] * alpha[:, None], dst, num_segments=N)

    node_mu, node_logvar = _dense_post(h_raw, gat_bias, W_nmu, b_nmu, W_nlv, b_nlv)

    xT = jnp.pad((x + noise2).T, ((0, 0), (0, N_PAD - N)))
    t = _pool(xT)
    attr_mu, attr_logvar = _attr_mlp(t, ln1_g, ln1_b, W1, b1, ln2_g, ln2_b,
                                     W2, b2, ln3_g, ln3_b, W3, b3,
                                     W_amu, b_amu, W_alv, b_alv)
    return (node_mu, node_logvar, attr_mu, attr_logvar)


# speedup vs baseline: 9.3376x; 5.9172x over previous
import functools

import jax
import jax.numpy as jnp
import numpy as np
from jax import lax
from jax.experimental import pallas as pl
from jax.experimental.pallas import tpu as pltpu
from jax.experimental.pallas import tpu_sc as plsc

N = 10000
E = 320000
D = 128
OUT = 128
MLP_IN = OUT * 8
H1 = OUT * 2
N_PAD = 10240


def _dense_pre_body(x_ref, wl_ref, bl_ref, wr_ref, br_ref, xl_ref, xr_ref):
    x = x_ref[...]
    xl_ref[...] = jnp.dot(x, wl_ref[...], preferred_element_type=jnp.float32) + bl_ref[...]
    xr_ref[...] = jnp.dot(x, wr_ref[...], preferred_element_type=jnp.float32) + br_ref[...]


def _dense_pre(x_eps, Wl, bl, Wr, br):
    return pl.pallas_call(
        _dense_pre_body,
        out_shape=(
            jax.ShapeDtypeStruct((N, OUT), jnp.float32),
            jax.ShapeDtypeStruct((N, OUT), jnp.float32),
        ),
    )(x_eps, Wl, bl.reshape(1, OUT), Wr, br.reshape(1, OUT))


def _dense_post_body(h0_ref, h1_ref, den_ref, gb_ref, wmu_ref, bmu_ref,
                     wlv_ref, blv_ref, mu_ref, lv_ref):
    inv = 1.0 / (den_ref[...] + 1e-16)
    h = jnp.maximum((h0_ref[...] + h1_ref[...]) * inv + gb_ref[...], 0.0)
    mu_ref[...] = jnp.dot(h, wmu_ref[...], preferred_element_type=jnp.float32) + bmu_ref[...]
    lv_ref[...] = jnp.dot(h, wlv_ref[...], preferred_element_type=jnp.float32) + blv_ref[...]


def _dense_post(h0, h1, den_col, gat_bias, W_nmu, b_nmu, W_nlv, b_nlv):
    return pl.pallas_call(
        _dense_post_body,
        out_shape=(
            jax.ShapeDtypeStruct((N_PAD, OUT), jnp.float32),
            jax.ShapeDtypeStruct((N_PAD, OUT), jnp.float32),
        ),
    )(h0, h1, den_col, gat_bias.reshape(1, OUT), W_nmu, b_nmu.reshape(1, OUT),
      W_nlv, b_nlv.reshape(1, OUT))


@functools.lru_cache(maxsize=1)
def _pool_matrix():
    i = np.arange(MLP_IN)
    starts = (i * N) // MLP_IN
    ends = ((i + 1) * N + MLP_IN - 1) // MLP_IN
    P = np.zeros((N_PAD, MLP_IN), np.float32)
    for j in range(MLP_IN):
        P[starts[j]:ends[j], j] = 1.0 / (ends[j] - starts[j])
    return jnp.asarray(P)


def _pool_body(xT_ref, p_ref, t_ref):
    @pl.when(pl.program_id(0) == 0)
    def _init():
        t_ref[...] = jnp.zeros_like(t_ref)
    t_ref[...] += jnp.dot(xT_ref[...], p_ref[...], preferred_element_type=jnp.float32)


def _pool(xT):
    KC = 8
    KB = N_PAD // KC
    return pl.pallas_call(
        _pool_body,
        grid=(KC,),
        in_specs=[
            pl.BlockSpec((D, KB), lambda k: (0, k)),
            pl.BlockSpec((KB, MLP_IN), lambda k: (k, 0)),
        ],
        out_specs=pl.BlockSpec((D, MLP_IN), lambda k: (0, 0)),
        out_shape=jax.ShapeDtypeStruct((D, MLP_IN), jnp.float32),
    )(xT, _pool_matrix())


def _ln(x, g, b):
    m = jnp.mean(x, axis=-1, keepdims=True)
    v = jnp.mean((x - m) ** 2, axis=-1, keepdims=True)
    return (x - m) * jax.lax.rsqrt(v + 1e-5) * g + b


def _attr_mlp_body(t_ref, ln1g, ln1b, w1, b1, ln2g, ln2b, w2, b2,
                   ln3g, ln3b, w3, b3, wamu, bamu, walv, balv,
                   mu_ref, lv_ref):
    t = t_ref[...]
    t = jnp.tanh(jnp.dot(_ln(t, ln1g[...], ln1b[...]), w1[...],
                         preferred_element_type=jnp.float32) + b1[...])
    t = jnp.tanh(jnp.dot(_ln(t, ln2g[...], ln2b[...]), w2[...],
                         preferred_element_type=jnp.float32) + b2[...])
    h_A = jnp.tanh(jnp.dot(_ln(t, ln3g[...], ln3b[...]), w3[...],
                           preferred_element_type=jnp.float32) + b3[...])
    mu_ref[...] = jnp.dot(h_A, wamu[...], preferred_element_type=jnp.float32) + bamu[...]
    lv_ref[...] = jnp.dot(h_A, walv[...], preferred_element_type=jnp.float32) + balv[...]


def _attr_mlp(t, ln1_g, ln1_b, W1, b1, ln2_g, ln2_b, W2, b2,
              ln3_g, ln3_b, W3, b3, W_amu, b_amu, W_alv, b_alv):
    r = lambda a: a.reshape(1, -1)
    return pl.pallas_call(
        _attr_mlp_body,
        out_shape=(
            jax.ShapeDtypeStruct((D, OUT), jnp.float32),
            jax.ShapeDtypeStruct((D, OUT), jnp.float32),
        ),
    )(t, r(ln1_g), r(ln1_b), W1, r(b1), r(ln2_g), r(ln2_b), W2, r(b2),
      r(ln3_g), r(ln3_b), W3, r(b3), W_amu, r(b_amu), W_alv, r(b_alv))


NW = 32
EW = E // NW
CH = 80
NCH = EW // CH
NROW = N_PAD // 128
STRIPE = N_PAD // 16


def _edge_body(xl_hbm, xr_hbm, src_hbm, dst_hbm, att_hbm, zer_hbm, zer1_hbm,
               h_out, den_out,
               att_v, zbuf, src_v, dst_v, xlr, xrr, den_loc, h_sh, sem):
    cid = lax.axis_index("c")
    sid = lax.axis_index("s")
    wid = cid * 16 + sid

    pltpu.sync_copy(att_hbm, att_v)
    pltpu.sync_copy(zer_hbm, zbuf)
    pltpu.sync_copy(zer1_hbm, den_loc)
    for j in range(STRIPE // 128):
        pltpu.sync_copy(zbuf, h_sh.at[pl.ds(sid * STRIPE + j * 128, 128)])
    plsc.subcore_barrier()

    lanes = lax.iota(jnp.int32, 16)
    perms = [lanes ^ s for s in (8, 4, 2, 1)]
    attc = [att_v[pl.ds(c * 16, 16)] for c in range(8)]
    ebase0 = wid * EW

    def chunk_body(i, _):
        ebase = ebase0 + i * CH
        pltpu.sync_copy(src_hbm.at[pl.ds(ebase, CH)], src_v)
        pltpu.sync_copy(dst_hbm.at[pl.ds(ebase, CH)], dst_v)
        pltpu.async_copy(xl_hbm.at[src_v], xlr, sem).wait()
        pltpu.async_copy(xr_hbm.at[dst_v], xrr, sem).wait()

        def group_body(g, _):
            d16 = dst_v[pl.ds(g * 16, 16)]
            a16 = jnp.zeros((16,), jnp.float32)
            for u in range(16):
                row = g * 16 + u
                vl = [xlr[row, pl.ds(c * 16, 16)] for c in range(8)]
                acc = jnp.zeros((16,), jnp.float32)
                for c in range(8):
                    z = vl[c] + xrr[row, pl.ds(c * 16, 16)]
                    z = jnp.maximum(z, 0.2 * z)
                    acc = acc + z * attc[c]
                for p in perms:
                    acc = acc + acc.at[p].get(mode="promise_in_bounds")
                av = jnp.exp(acc)
                a16 = jnp.where(lanes == u, av, a16)
                for c in range(8):
                    xlr[row, pl.ds(c * 16, 16)] = vl[c] * av
            plsc.addupdate_scatter(den_loc, [d16], a16)
            return 0

        lax.fori_loop(0, CH // 16, group_body, 0)
        pltpu.sync_copy(xlr, h_sh.at[dst_v], add=True)
        return 0

    lax.fori_loop(0, NCH, chunk_body, 0)

    plsc.subcore_barrier()
    pltpu.sync_copy(h_sh.at[pl.ds(sid * STRIPE, STRIPE)],
                    h_out.at[cid, pl.ds(sid * STRIPE, STRIPE)])
    pltpu.sync_copy(den_loc, den_out.at[wid])


def _edge_sc(xl, xr, src, dst, att):
    import functools as _ft
    mesh = plsc.VectorSubcoreMesh(core_axis_name="c", subcore_axis_name="s")
    zeros = jnp.zeros((128, 128), jnp.float32)
    zeros1 = jnp.zeros((N_PAD,), jnp.float32)
    f = _ft.partial(
        pl.kernel,
        mesh=mesh,
        compiler_params=pltpu.CompilerParams(needs_layout_passes=False),
        out_type=[
            jax.ShapeDtypeStruct((2, N_PAD, OUT), jnp.float32),
            jax.ShapeDtypeStruct((NW, N_PAD), jnp.float32),
        ],
        scratch_types=[
            pltpu.VMEM((128,), jnp.float32),
            pltpu.VMEM((128, 128), jnp.float32),
            pltpu.VMEM((CH,), jnp.int32),
            pltpu.VMEM((CH,), jnp.int32),
            pltpu.VMEM((CH, OUT), jnp.float32),
            pltpu.VMEM((CH, OUT), jnp.float32),
            pltpu.VMEM((N_PAD,), jnp.float32),
            pltpu.VMEM_SHARED((N_PAD, OUT), jnp.float32),
            pltpu.SemaphoreType.DMA,
        ],
    )(_edge_body)
    return f(xl, xr, src, dst, att, zeros, zeros1)


def kernel(x, edge_index, Wl, bl, Wr, br, att, gat_bias, W_nmu, b_nmu,
           W_nlv, b_nlv, ln1_g, ln1_b, W1, b1, ln2_g, ln2_b, W2, b2,
           ln3_g, ln3_b, W3, b3, W_amu, b_amu, W_alv, b_alv):
    nkey = jax.random.key(42)
    noise1 = jax.random.normal(jax.random.fold_in(nkey, 0), x.shape, x.dtype)
    noise2 = jax.random.normal(jax.random.fold_in(nkey, 1), x.shape, x.dtype)

    x_eps = x + noise1
    xl, xr = _dense_pre(x_eps, Wl, bl, Wr, br)

    src = edge_index[0]
    dst = edge_index[1]

    h_parts, den_parts = _edge_sc(xl, xr, src, dst, att)
    den_col = den_parts.sum(0).reshape(N_PAD, 1)

    node_mu, node_logvar = _dense_post(h_parts[0], h_parts[1], den_col,
                                       gat_bias, W_nmu, b_nmu, W_nlv, b_nlv)
    node_mu = node_mu[:N]
    node_logvar = node_logvar[:N]

    xT = jnp.pad((x + noise2).T, ((0, 0), (0, N_PAD - N)))
    t = _pool(xT)
    attr_mu, attr_logvar = _attr_mlp(t, ln1_g, ln1_b, W1, b1, ln2_g, ln2_b,
                                     W2, b2, ln3_g, ln3_b, W3, b3,
                                     W_amu, b_amu, W_alv, b_alv)
    return (node_mu, node_logvar, attr_mu, attr_logvar)


# speedup vs baseline: 12.8824x; 1.3796x over previous
import functools

import jax
import jax.numpy as jnp
import numpy as np
from jax import lax
from jax.experimental import pallas as pl
from jax.experimental.pallas import tpu as pltpu
from jax.experimental.pallas import tpu_sc as plsc

N = 10000
E = 320000
D = 128
OUT = 128
MLP_IN = OUT * 8
H1 = OUT * 2
N_PAD = 10240


def _dense_pre_body(x_ref, wl_ref, bl_ref, wr_ref, br_ref, xl_ref, xr_ref):
    x = x_ref[...]
    xl_ref[...] = jnp.dot(x, wl_ref[...], preferred_element_type=jnp.float32) + bl_ref[...]
    xr_ref[...] = jnp.dot(x, wr_ref[...], preferred_element_type=jnp.float32) + br_ref[...]


def _dense_pre(x_eps, Wl, bl, Wr, br):
    return pl.pallas_call(
        _dense_pre_body,
        out_shape=(
            jax.ShapeDtypeStruct((N, OUT), jnp.float32),
            jax.ShapeDtypeStruct((N, OUT), jnp.float32),
        ),
    )(x_eps, Wl, bl.reshape(1, OUT), Wr, br.reshape(1, OUT))


def _dense_post_body(h0_ref, h1_ref, den_ref, gb_ref, wmu_ref, bmu_ref,
                     wlv_ref, blv_ref, mu_ref, lv_ref):
    inv = 1.0 / (den_ref[...] + 1e-16)
    h = jnp.maximum((h0_ref[...] + h1_ref[...]) * inv + gb_ref[...], 0.0)
    mu_ref[...] = jnp.dot(h, wmu_ref[...], preferred_element_type=jnp.float32) + bmu_ref[...]
    lv_ref[...] = jnp.dot(h, wlv_ref[...], preferred_element_type=jnp.float32) + blv_ref[...]


def _dense_post(h0, h1, den_col, gat_bias, W_nmu, b_nmu, W_nlv, b_nlv):
    return pl.pallas_call(
        _dense_post_body,
        out_shape=(
            jax.ShapeDtypeStruct((N, OUT), jnp.float32),
            jax.ShapeDtypeStruct((N, OUT), jnp.float32),
        ),
    )(h0, h1, den_col, gat_bias.reshape(1, OUT), W_nmu, b_nmu.reshape(1, OUT),
      W_nlv, b_nlv.reshape(1, OUT))


@functools.lru_cache(maxsize=1)
def _pool_matrix():
    i = np.arange(MLP_IN)
    starts = (i * N) // MLP_IN
    ends = ((i + 1) * N + MLP_IN - 1) // MLP_IN
    P = np.zeros((N_PAD, MLP_IN), np.float32)
    for j in range(MLP_IN):
        P[starts[j]:ends[j], j] = 1.0 / (ends[j] - starts[j])
    return jnp.asarray(P)


def _pool_body(xT_ref, p_ref, t_ref):
    @pl.when(pl.program_id(0) == 0)
    def _init():
        t_ref[...] = jnp.zeros_like(t_ref)
    t_ref[...] += jnp.dot(xT_ref[...], p_ref[...], preferred_element_type=jnp.float32)


def _pool(xT):
    KC = 8
    KB = N_PAD // KC
    return pl.pallas_call(
        _pool_body,
        grid=(KC,),
        in_specs=[
            pl.BlockSpec((D, KB), lambda k: (0, k)),
            pl.BlockSpec((KB, MLP_IN), lambda k: (k, 0)),
        ],
        out_specs=pl.BlockSpec((D, MLP_IN), lambda k: (0, 0)),
        out_shape=jax.ShapeDtypeStruct((D, MLP_IN), jnp.float32),
    )(xT, _pool_matrix())


def _ln(x, g, b):
    m = jnp.mean(x, axis=-1, keepdims=True)
    v = jnp.mean((x - m) ** 2, axis=-1, keepdims=True)
    return (x - m) * jax.lax.rsqrt(v + 1e-5) * g + b


def _attr_mlp_body(t_ref, ln1g, ln1b, w1, b1, ln2g, ln2b, w2, b2,
                   ln3g, ln3b, w3, b3, wamu, bamu, walv, balv,
                   mu_ref, lv_ref):
    t = t_ref[...]
    t = jnp.tanh(jnp.dot(_ln(t, ln1g[...], ln1b[...]), w1[...],
                         preferred_element_type=jnp.float32) + b1[...])
    t = jnp.tanh(jnp.dot(_ln(t, ln2g[...], ln2b[...]), w2[...],
                         preferred_element_type=jnp.float32) + b2[...])
    h_A = jnp.tanh(jnp.dot(_ln(t, ln3g[...], ln3b[...]), w3[...],
                           preferred_element_type=jnp.float32) + b3[...])
    mu_ref[...] = jnp.dot(h_A, wamu[...], preferred_element_type=jnp.float32) + bamu[...]
    lv_ref[...] = jnp.dot(h_A, walv[...], preferred_element_type=jnp.float32) + balv[...]


def _attr_mlp(t, ln1_g, ln1_b, W1, b1, ln2_g, ln2_b, W2, b2,
              ln3_g, ln3_b, W3, b3, W_amu, b_amu, W_alv, b_alv):
    r = lambda a: a.reshape(1, -1)
    return pl.pallas_call(
        _attr_mlp_body,
        out_shape=(
            jax.ShapeDtypeStruct((D, OUT), jnp.float32),
            jax.ShapeDtypeStruct((D, OUT), jnp.float32),
        ),
    )(t, r(ln1_g), r(ln1_b), W1, r(b1), r(ln2_g), r(ln2_b), W2, r(b2),
      r(ln3_g), r(ln3_b), W3, r(b3), W_amu, r(b_amu), W_alv, r(b_alv))


NW = 32
EW = E // NW
CH = 48
NCHM = 208
TAIL = EW - NCHM * CH
STRIPE = 624


def _edge_body(xl_hbm, xr_hbm, src_hbm, dst_hbm, att_hbm, z2_hbm, z1_hbm,
               h_out, den_out,
               att_v, den_loc,
               src0, dst0, src1, dst1, src2, dst2,
               xlr0, xrr0, xlr1, xrr1, xlr2, xrr2, h_sh,
               si0, si1, si2, sg0, sg1, sg2, ss0, ss1, ss2):
    cid = lax.axis_index("c")
    sid = lax.axis_index("s")
    wid = cid * 16 + sid
    ebase0 = pl.multiple_of(wid * EW, 8)
    hbase = pl.multiple_of(sid * STRIPE, 8)
    SRC = [src0, src1, src2]
    DST = [dst0, dst1, dst2]
    XL = [xlr0, xlr1, xlr2]
    XR = [xrr0, xrr1, xrr2]
    SI = [si0, si1, si2]
    SG = [sg0, sg1, sg2]
    SS = [ss0, ss1, ss2]

    pltpu.sync_copy(att_hbm, att_v)
    pltpu.sync_copy(z1_hbm, den_loc)
    for j in range(4):
        pltpu.sync_copy(z2_hbm, h_sh.at[pl.ds(hbase + j * 128, 128)])
    pltpu.sync_copy(z2_hbm.at[pl.ds(0, STRIPE - 512)],
                    h_sh.at[pl.ds(hbase + 512, STRIPE - 512)])

    @pl.when(sid == 15)
    def _zrem():
        pltpu.sync_copy(z2_hbm.at[pl.ds(0, 16)], h_sh.at[pl.ds(16 * STRIPE, 16)])

    plsc.subcore_barrier()

    lanes = lax.iota(jnp.int32, 16)
    perms = [lanes ^ s for s in (8, 4, 2, 1)]
    attc = [att_v[pl.ds(c * 16, 16)] for c in range(8)]

    def idx_issue(i, b):
        o = pl.multiple_of(ebase0 + i * CH, 8)
        pltpu.async_copy(src_hbm.at[pl.ds(o, CH)], SRC[b], SI[b])
        pltpu.async_copy(dst_hbm.at[pl.ds(o, CH)], DST[b], SI[b])

    def wait_idx(b):
        pltpu.make_async_copy(src_hbm.at[pl.ds(0, CH)], SRC[b], SI[b]).wait()
        pltpu.make_async_copy(dst_hbm.at[pl.ds(0, CH)], DST[b], SI[b]).wait()

    def gather_issue(b):
        pltpu.async_copy(xl_hbm.at[SRC[b]], XL[b], SG[b])
        pltpu.async_copy(xr_hbm.at[DST[b]], XR[b], SG[b])

    def wait_gather(b):
        pltpu.make_async_copy(xl_hbm.at[SRC[b]], XL[b], SG[b]).wait()
        pltpu.make_async_copy(xr_hbm.at[DST[b]], XR[b], SG[b]).wait()

    def wait_scatter(b, rows=CH):
        pltpu.make_async_copy(xl_hbm.at[pl.ds(0, rows)], XL[b].at[pl.ds(0, rows)],
                              SS[b]).wait()

    def group(xlr, xrr, d16, g):
        def edge_body(u, a16):
            row = g * 16 + u
            vl = [xlr[row, pl.ds(c * 16, 16)] for c in range(8)]
            acc = jnp.zeros((16,), jnp.float32)
            for c in range(8):
                z = vl[c] + xrr[row, pl.ds(c * 16, 16)]
                z = jnp.maximum(z, 0.2 * z)
                acc = acc + z * attc[c]
            for p in perms:
                acc = acc + acc.at[p].get(mode="promise_in_bounds")
            av = jnp.exp(acc)
            a16 = jnp.where(lanes == u, av, a16)
            for c in range(8):
                xlr[row, pl.ds(c * 16, 16)] = vl[c] * av
            return a16

        a16 = lax.fori_loop(0, 16, edge_body, jnp.zeros((16,), jnp.float32))
        plsc.addupdate_scatter(den_loc, [d16], a16)

    def compute(b):
        xlr, xrr = XL[b], XR[b]

        def group_body(g, _):
            group(xlr, xrr, DST[b][pl.ds(g * 16, 16)], g)
            return 0

        lax.fori_loop(0, CH // 16, group_body, 0)
        pltpu.async_copy(xlr, h_sh.at[DST[b]], SS[b], add=True)

    idx_issue(0, 0)
    idx_issue(1, 1)
    wait_idx(0)
    gather_issue(0)
    wait_idx(1)
    gather_issue(1)
    idx_issue(2, 2)

    def outer(j, _):
        for b in range(3):
            i = j * 3 + b
            wait_gather(b)
            compute(b)
            bp = (b + 2) % 3
            if b == 0:
                @pl.when(j > 0)
                def _ws():
                    wait_scatter(bp)
            else:
                wait_scatter(bp)

            @pl.when(i + 2 < NCHM)
            def _g():
                wait_idx(bp)
                gather_issue(bp)

            @pl.when(i + 3 < NCHM)
            def _ii():
                idx_issue(i + 3, b)
        return 0

    lax.fori_loop(0, (NCHM - 1) // 3, outer, 0)

    wait_gather(0)
    compute(0)
    wait_scatter(2)

    ot = pl.multiple_of(ebase0 + NCHM * CH, 8)
    pltpu.sync_copy(src_hbm.at[pl.ds(ot, TAIL)], src1.at[pl.ds(0, TAIL)])
    pltpu.sync_copy(dst_hbm.at[pl.ds(ot, TAIL)], dst1.at[pl.ds(0, TAIL)])
    pltpu.async_copy(xl_hbm.at[src1.at[pl.ds(0, TAIL)]], xlr1.at[pl.ds(0, TAIL)], sg1)
    pltpu.async_copy(xr_hbm.at[dst1.at[pl.ds(0, TAIL)]], xrr1.at[pl.ds(0, TAIL)], sg1)
    pltpu.make_async_copy(xl_hbm.at[pl.ds(0, TAIL)], xlr1.at[pl.ds(0, TAIL)], sg1).wait()
    pltpu.make_async_copy(xl_hbm.at[pl.ds(0, TAIL)], xrr1.at[pl.ds(0, TAIL)], sg1).wait()
    group(xlr1, xrr1, dst1[pl.ds(0, 16)], 0)
    pltpu.async_copy(xlr1.at[pl.ds(0, TAIL)], h_sh.at[dst1.at[pl.ds(0, TAIL)]],
                     ss1, add=True)
    wait_scatter(0)
    wait_scatter(1, rows=TAIL)

    plsc.subcore_barrier()
    pltpu.sync_copy(h_sh.at[pl.ds(hbase, STRIPE)],
                    h_out.at[cid, pl.ds(hbase, STRIPE)])

    @pl.when(sid == 15)
    def _orem():
        pltpu.sync_copy(h_sh.at[pl.ds(16 * STRIPE, 16)],
                        h_out.at[cid, pl.ds(16 * STRIPE, 16)])

    pltpu.sync_copy(den_loc, den_out.at[wid])


def _edge_sc(xl, xr, src, dst, att):
    import functools as _ft
    mesh = plsc.VectorSubcoreMesh(core_axis_name="c", subcore_axis_name="s")
    zeros2 = jnp.zeros((128, 128), jnp.float32)
    zeros1 = jnp.zeros((N,), jnp.float32)
    f = _ft.partial(
        pl.kernel,
        mesh=mesh,
        compiler_params=pltpu.CompilerParams(needs_layout_passes=False),
        out_type=[
            jax.ShapeDtypeStruct((2, N, OUT), jnp.float32),
            jax.ShapeDtypeStruct((NW, N), jnp.float32),
        ],
        scratch_types=(
            [
                pltpu.VMEM((128,), jnp.float32),
                pltpu.VMEM((N,), jnp.float32),
            ]
            + [pltpu.VMEM((CH,), jnp.int32)] * 6
            + [pltpu.VMEM((CH, OUT), jnp.float32)] * 6
            + [pltpu.VMEM_SHARED((N, OUT), jnp.float32)]
            + [pltpu.SemaphoreType.DMA] * 9
        ),
    )(_edge_body)
    return f(xl, xr, src, dst, att, zeros2, zeros1)


def kernel(x, edge_index, Wl, bl, Wr, br, att, gat_bias, W_nmu, b_nmu,
           W_nlv, b_nlv, ln1_g, ln1_b, W1, b1, ln2_g, ln2_b, W2, b2,
           ln3_g, ln3_b, W3, b3, W_amu, b_amu, W_alv, b_alv):
    nkey = jax.random.key(42)
    noise1 = jax.random.normal(jax.random.fold_in(nkey, 0), x.shape, x.dtype)
    noise2 = jax.random.normal(jax.random.fold_in(nkey, 1), x.shape, x.dtype)

    x_eps = x + noise1
    xl, xr = _dense_pre(x_eps, Wl, bl, Wr, br)

    src = edge_index[0]
    dst = edge_index[1]

    h_parts, den_parts = _edge_sc(xl, xr, src, dst, att)
    den_col = den_parts.sum(0).reshape(N, 1)

    node_mu, node_logvar = _dense_post(h_parts[0], h_parts[1], den_col,
                                       gat_bias, W_nmu, b_nmu, W_nlv, b_nlv)

    xT = jnp.pad((x + noise2).T, ((0, 0), (0, N_PAD - N)))
    t = _pool(xT)
    attr_mu, attr_logvar = _attr_mlp(t, ln1_g, ln1_b, W1, b1, ln2_g, ln2_b,
                                     W2, b2, ln3_g, ln3_b, W3, b3,
                                     W_amu, b_amu, W_alv, b_alv)
    return (node_mu, node_logvar, attr_mu, attr_logvar)
